# baseline (speedup 1.0000x reference)
import jax
import jax.numpy as jnp
from jax.experimental import pallas as pl
from jax.experimental.pallas import tpu as pltpu

_GRID = 2


def _copy_body(x_hbm, b_ref, xo_ref, bo_ref, sem):
    i = pl.program_id(0)
    rows = xo_ref.shape[0]
    cp = pltpu.make_async_copy(x_hbm.at[pl.ds(i * rows, rows), :], xo_ref, sem)
    cp.start()

    @pl.when(i == 0)
    def _():
        bo_ref[...] = b_ref[...]

    cp.wait()


def kernel(x, edge_index, batch):
    del edge_index
    n, d = x.shape
    rows = n // _GRID
    xo, bo = pl.pallas_call(
        _copy_body,
        grid=(_GRID,),
        in_specs=[
            pl.BlockSpec(memory_space=pltpu.MemorySpace.HBM),
            pl.BlockSpec(batch.shape, lambda i: (0,)),
        ],
        out_specs=(
            pl.BlockSpec((rows, d), lambda i: (i, 0)),
            pl.BlockSpec(batch.shape, lambda i: (0,)),
        ),
        out_shape=(
            jax.ShapeDtypeStruct(x.shape, x.dtype),
            jax.ShapeDtypeStruct(batch.shape, batch.dtype),
        ),
        scratch_shapes=[pltpu.SemaphoreType.DMA],
    )(x, batch)
    return (xo, bo)
